# Initial kernel scaffold; baseline (speedup 1.0000x reference)
import jax
import jax.numpy as jnp
from jax.experimental import pallas as pl


def kernel(input_ids, bbox, token_type_ids, word_emb, pos_emb, x_emb, y_emb, h_emb, w_emb, tok_emb, ln_g, ln_b):
    raise NotImplementedError("write your pallas kernel here")


# speedup vs baseline: 1.3428x; 1.3428x over previous
import functools

import jax
import jax.numpy as jnp
from jax import lax
from jax.experimental import pallas as pl
from jax.experimental.pallas import tpu as pltpu
from jax.experimental.pallas import tpu_sc as plsc

_B, _S, _H = 1024, 200, 768
_N = _B * _S
_NC, _NS = 2, 16
_NW = _NC * _NS
_TPW = _N // _NW
_C = 16
_NCHUNK = _TPW // _C
_NV = _H // 16
_EPS = 1e-12
_NT = 9


def _rsqrt(x):
    i = lax.bitcast_convert_type(x, jnp.int32)
    i = jnp.int32(0x5F3759DF) - lax.shift_right_logical(i, 1)
    y = lax.bitcast_convert_type(i, jnp.float32)
    half = jnp.float32(0.5) * x
    for _ in range(3):
        y = y * (jnp.float32(1.5) - half * y * y)
    return y


def _sc_body(idx_hbm, word_hbm, pos_hbm, x_hbm, y_hbm, h_hbm, w_hbm,
             tok_hbm, g_hbm, b_hbm, out_hbm, idxv, bufs, gv, bv, red, sem):
    wid = lax.axis_index("s") * _NC + lax.axis_index("c")
    base = wid * _TPW
    pltpu.sync_copy(g_hbm, gv)
    pltpu.sync_copy(b_hbm, bv)
    tables = (word_hbm, pos_hbm, x_hbm, y_hbm, x_hbm, y_hbm, h_hbm,
              w_hbm, tok_hbm)

    def chunk_body(g, carry):
        off = base + g * _C
        pltpu.sync_copy(idx_hbm.at[wid, g], idxv)
        cps = [pltpu.async_copy(tables[t].at[idxv.at[t]], bufs.at[t], sem)
               for t in range(_NT)]
        for cp in cps:
            cp.wait()

        def tok_body(tk, c):
            vs = jnp.zeros((16,), jnp.float32)
            vq = jnp.zeros((16,), jnp.float32)
            for j in range(_NV):
                sl = pl.ds(j * 16, 16)
                a = bufs[0, tk, sl]
                for t in range(1, _NT):
                    a = a + bufs[t, tk, sl]
                bufs[0, tk, sl] = a
                vs = vs + a
                vq = vq + a * a
            s = vs[0]
            q = vq[0]
            for i in range(1, 16):
                s = s + vs[i]
                q = q + vq[i]
            mean = s * jnp.float32(1.0 / _H)
            m2 = q * jnp.float32(1.0 / _H)
            rstd = _rsqrt(m2 - mean * mean + jnp.float32(_EPS))
            for j in range(_NV):
                sl = pl.ds(j * 16, 16)
                a = bufs[0, tk, sl]
                bufs[0, tk, sl] = (a - mean) * rstd * gv[sl] + bv[sl]
            return c

        lax.fori_loop(0, _C, tok_body, 0)
        pltpu.sync_copy(bufs.at[0], out_hbm.at[pl.ds(off, _C)])
        return carry

    lax.fori_loop(0, _NCHUNK, chunk_body, 0)


@jax.jit
def _run(idx_all, word_emb, pos_emb, x_emb, y_emb, h_emb, w_emb, tok_emb,
         ln_g, ln_b):
    mesh = plsc.VectorSubcoreMesh(core_axis_name="c", subcore_axis_name="s")
    f = pl.kernel(
        _sc_body,
        out_type=jax.ShapeDtypeStruct((_N, _H), jnp.float32),
        mesh=mesh,
        scratch_types=[
            pltpu.VMEM((_NT, _C), jnp.int32),
            pltpu.VMEM((_NT, _C, _H), jnp.float32),
            pltpu.VMEM((_H,), jnp.float32),
            pltpu.VMEM((_H,), jnp.float32),
            pltpu.VMEM((2, 16), jnp.float32),
            pltpu.SemaphoreType.DMA,
        ],
    )
    return f(idx_all, word_emb, pos_emb, x_emb, y_emb, h_emb, w_emb,
             tok_emb, ln_g, ln_b)


def kernel(input_ids, bbox, token_type_ids, word_emb, pos_emb, x_emb, y_emb,
           h_emb, w_emb, tok_emb, ln_g, ln_b):
    ids = input_ids.reshape(_N).astype(jnp.int32)
    tts = token_type_ids.reshape(_N).astype(jnp.int32)
    bb = bbox.reshape(_N, 4).astype(jnp.int32)
    pos = jnp.broadcast_to(jnp.arange(_S, dtype=jnp.int32), (_B, _S))
    pos = pos.reshape(_N)
    idx_all = jnp.stack([
        ids, pos, bb[:, 0], bb[:, 1], bb[:, 2], bb[:, 3],
        bb[:, 3] - bb[:, 1], bb[:, 2] - bb[:, 0], tts,
    ])
    idx_all = idx_all.reshape(_NT, _NW, _NCHUNK, _C).transpose(1, 2, 0, 3)
    out = _run(idx_all, word_emb, pos_emb, x_emb, y_emb, h_emb, w_emb,
               tok_emb, ln_g, ln_b)
    return out.reshape(_B, _S, _H)
